# baseline (speedup 1.0000x reference)
import functools

import jax
import jax.numpy as jnp
from jax import lax
from jax.experimental import pallas as pl
from jax.experimental.pallas import tpu as pltpu
from jax.experimental.pallas import tpu_sc as plsc

K = 2
BT = 512
NEG = jnp.float32(-3e38)


def _gating_body(x_ref, wg_ref, xbf_ref, logits_ref):
    xb = x_ref[...]
    xbf_ref[...] = xb.astype(jnp.bfloat16)
    logits_ref[...] = lax.dot_general(
        xb, wg_ref[...], (((1,), (0,)), ((), ())),
        precision=lax.Precision.HIGHEST,
        preferred_element_type=jnp.float32)


def _moe_body(xbf_ref, gates_ref, w1_ref, b1_ref, g1_ref, be1_ref, w2_ref,
              b2_ref, y_ref):
    xb = xbf_ref[...]
    gates = gates_ref[...]
    num_e = gates.shape[1]
    acc = jnp.zeros((xb.shape[0], 1), jnp.float32)
    for e in range(num_e):
        h = lax.dot_general(
            xb, w1_ref[e], (((1,), (0,)), ((), ())),
            preferred_element_type=jnp.float32)
        h = h + b1_ref[e:e + 1, :]
        mu = jnp.mean(h, axis=1, keepdims=True)
        d = h - mu
        var = jnp.mean(d * d, axis=1, keepdims=True)
        hn = d * lax.rsqrt(var + 1e-5) * g1_ref[e:e + 1, :] + be1_ref[e:e + 1, :]
        hr = jnp.maximum(hn, 0.0)
        z = jnp.sum(hr * w2_ref[e:e + 1, :], axis=1, keepdims=True)
        o = jax.nn.sigmoid(z + b2_ref[0:1, e:e + 1])
        acc = acc + gates[:, e:e + 1] * o
    y_ref[...] = acc


def _make_gate_sc(bsz, num_e):
    info = plsc.get_sparse_core_info()
    nc, ns, nl = info.num_cores, info.num_subcores, info.num_lanes
    nw = nc * ns
    chunk = bsz // nw
    mesh = plsc.VectorSubcoreMesh(core_axis_name="c", subcore_axis_name="s")

    @functools.partial(
        pl.kernel, mesh=mesh,
        out_type=jax.ShapeDtypeStruct((num_e * bsz,), jnp.float32),
        scratch_types=[pltpu.VMEM((num_e * chunk,), jnp.float32),
                       pltpu.VMEM((num_e * chunk,), jnp.float32)],
    )
    def gate_sc(logits_hbm, gates_hbm, lg_v, out_v):
        wid = lax.axis_index("s") * nc + lax.axis_index("c")
        base = wid * chunk
        for e in range(num_e):
            pltpu.sync_copy(logits_hbm.at[pl.ds(e * bsz + base, chunk)],
                            lg_v.at[pl.ds(e * chunk, chunk)])
        for g in range(chunk // nl):
            l = [lg_v[pl.ds(e * chunk + g * nl, nl)] for e in range(num_e)]
            m1 = l[0]
            for e in range(1, num_e):
                m1 = jnp.maximum(m1, l[e])
            idx1 = jnp.full((nl,), num_e - 1, jnp.int32)
            for e in range(num_e - 2, -1, -1):
                idx1 = jnp.where(l[e] == m1, e, idx1)
            lm = [jnp.where(idx1 == e, NEG, l[e]) for e in range(num_e)]
            m2 = lm[0]
            for e in range(1, num_e):
                m2 = jnp.maximum(m2, lm[e])
            idx2 = jnp.full((nl,), num_e - 1, jnp.int32)
            for e in range(num_e - 2, -1, -1):
                idx2 = jnp.where(lm[e] == m2, e, idx2)
            s = jnp.exp(l[0] - m1)
            for e in range(1, num_e):
                s = s + jnp.exp(l[e] - m1)
            p1 = 1.0 / s
            p2 = jnp.exp(m2 - m1) / s
            denom = p1 + p2 + 1e-6
            g1 = p1 / denom
            g2 = p2 / denom
            zero = jnp.zeros((nl,), jnp.float32)
            for e in range(num_e):
                ge = jnp.where(idx1 == e, g1, jnp.where(idx2 == e, g2, zero))
                out_v[pl.ds(e * chunk + g * nl, nl)] = ge
        for e in range(num_e):
            pltpu.sync_copy(out_v.at[pl.ds(e * chunk, chunk)],
                            gates_hbm.at[pl.ds(e * bsz + base, chunk)])

    return gate_sc


def kernel(x, w_gate, W1, b1, gamma1, beta1, W2, b2):
    bsz, d = x.shape
    num_e = w_gate.shape[1]
    hdim = W1.shape[2]
    grid = bsz // BT

    xbf, logits = pl.pallas_call(
        _gating_body,
        grid=(grid,),
        in_specs=[
            pl.BlockSpec((BT, d), lambda i: (i, 0)),
            pl.BlockSpec((d, num_e), lambda i: (0, 0)),
        ],
        out_specs=[
            pl.BlockSpec((BT, d), lambda i: (i, 0)),
            pl.BlockSpec((BT, num_e), lambda i: (i, 0)),
        ],
        out_shape=[
            jax.ShapeDtypeStruct((bsz, d), jnp.bfloat16),
            jax.ShapeDtypeStruct((bsz, num_e), jnp.float32),
        ],
        compiler_params=pltpu.CompilerParams(
            dimension_semantics=("arbitrary",)),
    )(x, w_gate)

    gates = _make_gate_sc(bsz, num_e)(
        logits.T.reshape(num_e * bsz)).reshape(num_e, bsz).T

    w1bf = W1.astype(jnp.bfloat16)
    w2s = W2[:, :, 0]
    b2r = b2.reshape(1, num_e)

    y = pl.pallas_call(
        _moe_body,
        grid=(grid,),
        in_specs=[
            pl.BlockSpec((BT, d), lambda i: (i, 0)),
            pl.BlockSpec((BT, num_e), lambda i: (i, 0)),
            pl.BlockSpec((num_e, d, hdim), lambda i: (0, 0, 0)),
            pl.BlockSpec((num_e, hdim), lambda i: (0, 0)),
            pl.BlockSpec((num_e, hdim), lambda i: (0, 0)),
            pl.BlockSpec((num_e, hdim), lambda i: (0, 0)),
            pl.BlockSpec((num_e, hdim), lambda i: (0, 0)),
            pl.BlockSpec((1, num_e), lambda i: (0, 0)),
        ],
        out_specs=pl.BlockSpec((BT, 1), lambda i: (i, 0)),
        out_shape=jax.ShapeDtypeStruct((bsz, 1), jnp.float32),
        compiler_params=pltpu.CompilerParams(
            dimension_semantics=("arbitrary",)),
    )(xbf, gates, w1bf, b1, gamma1, beta1, w2s, b2r)

    return y


# speedup vs baseline: 1.0845x; 1.0845x over previous
import functools

import jax
import jax.numpy as jnp
from jax import lax
from jax.experimental import pallas as pl
from jax.experimental.pallas import tpu as pltpu
from jax.experimental.pallas import tpu_sc as plsc

K = 2
BT = 512
NEG = -3e38


def _gating_body(x_ref, wg_ref, xbf_ref, logits_ref):
    xb = x_ref[...]
    xh = xb.astype(jnp.bfloat16)
    xbf_ref[...] = xh
    xl = (xb - xh.astype(jnp.float32)).astype(jnp.bfloat16)
    wg = wg_ref[...]
    wh = wg.astype(jnp.bfloat16)
    wl = (wg - wh.astype(jnp.float32)).astype(jnp.bfloat16)
    dims = (((1,), (0,)), ((), ()))
    acc = lax.dot_general(xh, wh, dims, preferred_element_type=jnp.float32)
    acc = acc + lax.dot_general(xl, wh, dims,
                                preferred_element_type=jnp.float32)
    acc = acc + lax.dot_general(xh, wl, dims,
                                preferred_element_type=jnp.float32)
    logits_ref[...] = acc


def _moe_body(xbf_ref, w1_ref, b1_ref, g1_ref, be1_ref, w2_ref, b2_ref, o_ref):
    xb = xbf_ref[...]
    num_e = w1_ref.shape[0]
    cols = []
    for e in range(num_e):
        h = lax.dot_general(
            xb, w1_ref[e], (((1,), (0,)), ((), ())),
            preferred_element_type=jnp.float32)
        h = h + b1_ref[e:e + 1, :]
        mu = jnp.mean(h, axis=1, keepdims=True)
        d = h - mu
        var = jnp.mean(d * d, axis=1, keepdims=True)
        hn = d * lax.rsqrt(var + 1e-5) * g1_ref[e:e + 1, :] + be1_ref[e:e + 1, :]
        hr = jnp.maximum(hn, 0.0)
        z = jnp.sum(hr * w2_ref[e:e + 1, :], axis=1, keepdims=True)
        cols.append(jax.nn.sigmoid(z + b2_ref[0:1, e:e + 1]))
    o_ref[...] = jnp.concatenate(cols, axis=1)


def _combine_body(gates_ref, o_ref, y_ref):
    y_ref[...] = jnp.sum(gates_ref[...] * o_ref[...], axis=1, keepdims=True)


def _make_gate_sc(bsz, num_e):
    info = plsc.get_sparse_core_info()
    nc, ns, nl = info.num_cores, info.num_subcores, info.num_lanes
    nw = nc * ns
    chunk = bsz // nw
    mesh = plsc.VectorSubcoreMesh(core_axis_name="c", subcore_axis_name="s")

    @functools.partial(
        pl.kernel, mesh=mesh,
        out_type=jax.ShapeDtypeStruct((num_e * bsz,), jnp.float32),
        scratch_types=[pltpu.VMEM((num_e * chunk,), jnp.float32),
                       pltpu.VMEM((num_e * chunk,), jnp.float32)],
    )
    def gate_sc(logits_hbm, gates_hbm, lg_v, out_v):
        wid = lax.axis_index("s") * nc + lax.axis_index("c")
        base = wid * chunk
        for e in range(num_e):
            pltpu.sync_copy(logits_hbm.at[pl.ds(e * bsz + base, chunk)],
                            lg_v.at[pl.ds(e * chunk, chunk)])
        for g in range(chunk // nl):
            l = [lg_v[pl.ds(e * chunk + g * nl, nl)] for e in range(num_e)]
            m1 = l[0]
            for e in range(1, num_e):
                m1 = jnp.maximum(m1, l[e])
            idx1 = jnp.full((nl,), num_e - 1, jnp.int32)
            for e in range(num_e - 2, -1, -1):
                idx1 = jnp.where(l[e] == m1, e, idx1)
            lm = [jnp.where(idx1 == e, NEG, l[e]) for e in range(num_e)]
            m2 = lm[0]
            for e in range(1, num_e):
                m2 = jnp.maximum(m2, lm[e])
            idx2 = jnp.full((nl,), num_e - 1, jnp.int32)
            for e in range(num_e - 2, -1, -1):
                idx2 = jnp.where(lm[e] == m2, e, idx2)
            s = jnp.exp(l[0] - m1)
            for e in range(1, num_e):
                s = s + jnp.exp(l[e] - m1)
            p1 = 1.0 / s
            p2 = jnp.exp(m2 - m1) / s
            denom = p1 + p2 + 1e-6
            g1 = p1 / denom
            g2 = p2 / denom
            zero = jnp.zeros((nl,), jnp.float32)
            for e in range(num_e):
                ge = jnp.where(idx1 == e, g1, jnp.where(idx2 == e, g2, zero))
                out_v[pl.ds(e * chunk + g * nl, nl)] = ge
        for e in range(num_e):
            pltpu.sync_copy(out_v.at[pl.ds(e * chunk, chunk)],
                            gates_hbm.at[pl.ds(e * bsz + base, chunk)])

    return gate_sc


def kernel(x, w_gate, W1, b1, gamma1, beta1, W2, b2):
    bsz, d = x.shape
    num_e = w_gate.shape[1]
    hdim = W1.shape[2]
    grid = bsz // BT

    xbf, logits = pl.pallas_call(
        _gating_body,
        grid=(grid,),
        in_specs=[
            pl.BlockSpec((BT, d), lambda i: (i, 0)),
            pl.BlockSpec((d, num_e), lambda i: (0, 0)),
        ],
        out_specs=[
            pl.BlockSpec((BT, d), lambda i: (i, 0)),
            pl.BlockSpec((BT, num_e), lambda i: (i, 0)),
        ],
        out_shape=[
            jax.ShapeDtypeStruct((bsz, d), jnp.bfloat16),
            jax.ShapeDtypeStruct((bsz, num_e), jnp.float32),
        ],
        compiler_params=pltpu.CompilerParams(
            dimension_semantics=("arbitrary",)),
    )(x, w_gate)

    gates = _make_gate_sc(bsz, num_e)(
        logits.T.reshape(num_e * bsz)).reshape(num_e, bsz).T

    w1bf = W1.astype(jnp.bfloat16)
    w2s = W2[:, :, 0]
    b2r = b2.reshape(1, num_e)

    o = pl.pallas_call(
        _moe_body,
        grid=(grid,),
        in_specs=[
            pl.BlockSpec((BT, d), lambda i: (i, 0)),
            pl.BlockSpec((num_e, d, hdim), lambda i: (0, 0, 0)),
            pl.BlockSpec((num_e, hdim), lambda i: (0, 0)),
            pl.BlockSpec((num_e, hdim), lambda i: (0, 0)),
            pl.BlockSpec((num_e, hdim), lambda i: (0, 0)),
            pl.BlockSpec((num_e, hdim), lambda i: (0, 0)),
            pl.BlockSpec((1, num_e), lambda i: (0, 0)),
        ],
        out_specs=pl.BlockSpec((BT, num_e), lambda i: (i, 0)),
        out_shape=jax.ShapeDtypeStruct((bsz, num_e), jnp.float32),
        compiler_params=pltpu.CompilerParams(
            dimension_semantics=("arbitrary",)),
    )(xbf, w1bf, b1, gamma1, beta1, w2s, b2r)

    y = pl.pallas_call(
        _combine_body,
        in_specs=[
            pl.BlockSpec((bsz, num_e), lambda: (0, 0)),
            pl.BlockSpec((bsz, num_e), lambda: (0, 0)),
        ],
        out_specs=pl.BlockSpec((bsz, 1), lambda: (0, 0)),
        out_shape=jax.ShapeDtypeStruct((bsz, 1), jnp.float32),
    )(gates, o)

    return y


# speedup vs baseline: 1.4125x; 1.3024x over previous
import functools

import jax
import jax.numpy as jnp
from jax import lax
from jax.experimental import pallas as pl
from jax.experimental.pallas import tpu as pltpu
from jax.experimental.pallas import tpu_sc as plsc

K = 2
BT = 512
NEG = -3e38
DIMS_T = (((0,), (0,)), ((), ()))


def _gating_body(xt_ref, wg_ref, logits_ref):
    xt = xt_ref[...]
    xh = xt.astype(jnp.bfloat16)
    xl = (xt - xh.astype(jnp.float32)).astype(jnp.bfloat16)
    wg = wg_ref[...]
    wh = wg.astype(jnp.bfloat16)
    wl = (wg - wh.astype(jnp.float32)).astype(jnp.bfloat16)
    acc = lax.dot_general(xh, wh, DIMS_T, preferred_element_type=jnp.float32)
    acc = acc + lax.dot_general(xl, wh, DIMS_T,
                                preferred_element_type=jnp.float32)
    acc = acc + lax.dot_general(xh, wl, DIMS_T,
                                preferred_element_type=jnp.float32)
    logits_ref[...] = acc


def _moe_body(xt_ref, w1_ref, b1_ref, g1_ref, be1_ref, w2_ref, b2_ref,
              o_ref, w1bf_ref):
    i = pl.program_id(0)

    @pl.when(i == 0)
    def _():
        w1bf_ref[...] = w1_ref[...].astype(jnp.bfloat16)

    xb = xt_ref[...].astype(jnp.bfloat16)
    num_e = w1_ref.shape[0]
    cols = []
    for e in range(num_e):
        h = lax.dot_general(
            xb, w1bf_ref[e], DIMS_T,
            preferred_element_type=jnp.float32)
        h = h + b1_ref[e:e + 1, :]
        mu = jnp.mean(h, axis=1, keepdims=True)
        d = h - mu
        var = jnp.mean(d * d, axis=1, keepdims=True)
        hn = d * lax.rsqrt(var + 1e-5) * g1_ref[e:e + 1, :] + be1_ref[e:e + 1, :]
        hr = jnp.maximum(hn, 0.0)
        z = jnp.sum(hr * w2_ref[e:e + 1, :], axis=1, keepdims=True)
        cols.append(jax.nn.sigmoid(z + b2_ref[0:1, e:e + 1]))
    o_ref[...] = jnp.concatenate(cols, axis=1)


def _combine_body(gates_ref, o_ref, y_ref):
    y_ref[...] = jnp.sum(gates_ref[...] * o_ref[...], axis=1, keepdims=True)


def _make_gate_sc(bsz, num_e):
    info = plsc.get_sparse_core_info()
    nc, ns, nl = info.num_cores, info.num_subcores, info.num_lanes
    nw = nc * ns
    chunk = bsz // nw
    mesh = plsc.VectorSubcoreMesh(core_axis_name="c", subcore_axis_name="s")

    @functools.partial(
        pl.kernel, mesh=mesh,
        out_type=jax.ShapeDtypeStruct((num_e * bsz,), jnp.float32),
        scratch_types=[pltpu.VMEM((num_e * chunk,), jnp.float32),
                       pltpu.VMEM((num_e * chunk,), jnp.float32)],
    )
    def gate_sc(logits_hbm, gates_hbm, lg_v, out_v):
        wid = lax.axis_index("s") * nc + lax.axis_index("c")
        base = wid * chunk
        for e in range(num_e):
            pltpu.sync_copy(logits_hbm.at[pl.ds(e * bsz + base, chunk)],
                            lg_v.at[pl.ds(e * chunk, chunk)])
        for g in range(chunk // nl):
            l = [lg_v[pl.ds(e * chunk + g * nl, nl)] for e in range(num_e)]
            m1 = l[0]
            for e in range(1, num_e):
                m1 = jnp.maximum(m1, l[e])
            idx1 = jnp.full((nl,), num_e - 1, jnp.int32)
            for e in range(num_e - 2, -1, -1):
                idx1 = jnp.where(l[e] == m1, e, idx1)
            lm = [jnp.where(idx1 == e, NEG, l[e]) for e in range(num_e)]
            m2 = lm[0]
            for e in range(1, num_e):
                m2 = jnp.maximum(m2, lm[e])
            idx2 = jnp.full((nl,), num_e - 1, jnp.int32)
            for e in range(num_e - 2, -1, -1):
                idx2 = jnp.where(lm[e] == m2, e, idx2)
            s = jnp.exp(l[0] - m1)
            for e in range(1, num_e):
                s = s + jnp.exp(l[e] - m1)
            p1 = 1.0 / s
            p2 = jnp.exp(m2 - m1) / s
            denom = p1 + p2 + 1e-6
            g1 = p1 / denom
            g2 = p2 / denom
            zero = jnp.zeros((nl,), jnp.float32)
            for e in range(num_e):
                ge = jnp.where(idx1 == e, g1, jnp.where(idx2 == e, g2, zero))
                out_v[pl.ds(e * chunk + g * nl, nl)] = ge
        for e in range(num_e):
            pltpu.sync_copy(out_v.at[pl.ds(e * chunk, chunk)],
                            gates_hbm.at[pl.ds(e * bsz + base, chunk)])

    return gate_sc


def kernel(x, w_gate, W1, b1, gamma1, beta1, W2, b2):
    bsz, d = x.shape
    num_e = w_gate.shape[1]
    hdim = W1.shape[2]
    grid = bsz // BT
    xt = x.T

    logits = pl.pallas_call(
        _gating_body,
        grid=(grid,),
        in_specs=[
            pl.BlockSpec((d, BT), lambda i: (0, i)),
            pl.BlockSpec((d, num_e), lambda i: (0, 0)),
        ],
        out_specs=pl.BlockSpec((BT, num_e), lambda i: (i, 0)),
        out_shape=jax.ShapeDtypeStruct((bsz, num_e), jnp.float32),
        compiler_params=pltpu.CompilerParams(
            dimension_semantics=("arbitrary",)),
    )(xt, w_gate)

    gates = _make_gate_sc(bsz, num_e)(
        logits.T.reshape(num_e * bsz)).reshape(num_e, bsz).T

    w2s = W2[:, :, 0]
    b2r = b2.reshape(1, num_e)

    o = pl.pallas_call(
        _moe_body,
        grid=(grid,),
        in_specs=[
            pl.BlockSpec((d, BT), lambda i: (0, i)),
            pl.BlockSpec((num_e, d, hdim), lambda i: (0, 0, 0)),
            pl.BlockSpec((num_e, hdim), lambda i: (0, 0)),
            pl.BlockSpec((num_e, hdim), lambda i: (0, 0)),
            pl.BlockSpec((num_e, hdim), lambda i: (0, 0)),
            pl.BlockSpec((num_e, hdim), lambda i: (0, 0)),
            pl.BlockSpec((1, num_e), lambda i: (0, 0)),
        ],
        out_specs=pl.BlockSpec((BT, num_e), lambda i: (i, 0)),
        out_shape=jax.ShapeDtypeStruct((bsz, num_e), jnp.float32),
        scratch_shapes=[pltpu.VMEM((num_e, d, hdim), jnp.bfloat16)],
        compiler_params=pltpu.CompilerParams(
            dimension_semantics=("arbitrary",)),
    )(xt, W1, b1, gamma1, beta1, w2s, b2r)

    y = pl.pallas_call(
        _combine_body,
        in_specs=[
            pl.BlockSpec((bsz, num_e), lambda: (0, 0)),
            pl.BlockSpec((bsz, num_e), lambda: (0, 0)),
        ],
        out_specs=pl.BlockSpec((bsz, 1), lambda: (0, 0)),
        out_shape=jax.ShapeDtypeStruct((bsz, 1), jnp.float32),
    )(gates, o)

    return y


# speedup vs baseline: 1.4805x; 1.0482x over previous
import functools

import jax
import jax.numpy as jnp
from jax import lax
from jax.experimental import pallas as pl
from jax.experimental.pallas import tpu as pltpu
from jax.experimental.pallas import tpu_sc as plsc

K = 2
BT = 1024
BTG = 2048
NEG = -3e38
DIMS_T = (((0,), (0,)), ((), ()))
DIMS_TT = (((0,), (1,)), ((), ()))


def _gating_body(xt_ref, wgt_ref, logits_ref):
    xt = xt_ref[...]
    xh = xt.astype(jnp.bfloat16)
    xl = (xt - xh.astype(jnp.float32)).astype(jnp.bfloat16)
    wg = wgt_ref[...]
    wh = wg.astype(jnp.bfloat16)
    wl = (wg - wh.astype(jnp.float32)).astype(jnp.bfloat16)
    acc = lax.dot_general(xh, wh, DIMS_TT, preferred_element_type=jnp.float32)
    acc = acc + lax.dot_general(xl, wh, DIMS_TT,
                                preferred_element_type=jnp.float32)
    acc = acc + lax.dot_general(xh, wl, DIMS_TT,
                                preferred_element_type=jnp.float32)
    logits_ref[...] = acc


def _moe_body(xt_ref, w1_ref, b1_ref, g1_ref, be1_ref, w2_ref, b2_ref,
              o_ref, w1bf_ref):
    i = pl.program_id(0)

    @pl.when(i == 0)
    def _():
        w1bf_ref[...] = w1_ref[...].astype(jnp.bfloat16)

    xb = xt_ref[...].astype(jnp.bfloat16)
    num_e = w1_ref.shape[0]
    cols = []
    for e in range(num_e):
        h = lax.dot_general(
            xb, w1bf_ref[e], DIMS_T,
            preferred_element_type=jnp.float32)
        h = h + b1_ref[e:e + 1, :]
        mu = jnp.mean(h, axis=1, keepdims=True)
        d = h - mu
        var = jnp.mean(d * d, axis=1, keepdims=True)
        hn = d * lax.rsqrt(var + 1e-5) * g1_ref[e:e + 1, :] + be1_ref[e:e + 1, :]
        hr = jnp.maximum(hn, 0.0)
        z = jnp.sum(hr * w2_ref[e:e + 1, :], axis=1, keepdims=True)
        cols.append(jax.nn.sigmoid(z + b2_ref[0:1, e:e + 1]))
    o_ref[...] = jnp.concatenate(cols, axis=1)


def _combine_body(gates_ref, o_ref, y_ref):
    y_ref[...] = jnp.sum(gates_ref[...] * o_ref[...], axis=1, keepdims=True)


def _make_gate_sc(bsz, num_e):
    info = plsc.get_sparse_core_info()
    nc, ns, nl = info.num_cores, info.num_subcores, info.num_lanes
    nw = nc * ns
    chunk = bsz // nw
    mesh = plsc.VectorSubcoreMesh(core_axis_name="c", subcore_axis_name="s")

    @functools.partial(
        pl.kernel, mesh=mesh,
        out_type=jax.ShapeDtypeStruct((num_e * bsz,), jnp.float32),
        scratch_types=[pltpu.VMEM((num_e * chunk,), jnp.float32),
                       pltpu.VMEM((num_e * chunk,), jnp.float32)],
    )
    def gate_sc(logits_hbm, gates_hbm, lg_v, out_v):
        wid = lax.axis_index("s") * nc + lax.axis_index("c")
        base = wid * chunk
        for e in range(num_e):
            pltpu.sync_copy(logits_hbm.at[pl.ds(e * bsz + base, chunk)],
                            lg_v.at[pl.ds(e * chunk, chunk)])
        for g in range(chunk // nl):
            l = [lg_v[pl.ds(e * chunk + g * nl, nl)] for e in range(num_e)]
            m1 = l[0]
            for e in range(1, num_e):
                m1 = jnp.maximum(m1, l[e])
            idx1 = jnp.full((nl,), num_e - 1, jnp.int32)
            for e in range(num_e - 2, -1, -1):
                idx1 = jnp.where(l[e] == m1, e, idx1)
            lm = [jnp.where(idx1 == e, NEG, l[e]) for e in range(num_e)]
            m2 = lm[0]
            for e in range(1, num_e):
                m2 = jnp.maximum(m2, lm[e])
            idx2 = jnp.full((nl,), num_e - 1, jnp.int32)
            for e in range(num_e - 2, -1, -1):
                idx2 = jnp.where(lm[e] == m2, e, idx2)
            s = jnp.exp(l[0] - m1)
            for e in range(1, num_e):
                s = s + jnp.exp(l[e] - m1)
            p1 = 1.0 / s
            p2 = jnp.exp(m2 - m1) / s
            denom = p1 + p2 + 1e-6
            g1 = p1 / denom
            g2 = p2 / denom
            zero = jnp.zeros((nl,), jnp.float32)
            for e in range(num_e):
                ge = jnp.where(idx1 == e, g1, jnp.where(idx2 == e, g2, zero))
                out_v[pl.ds(e * chunk + g * nl, nl)] = ge
        for e in range(num_e):
            pltpu.sync_copy(out_v.at[pl.ds(e * chunk, chunk)],
                            gates_hbm.at[pl.ds(e * bsz + base, chunk)])

    return gate_sc


def kernel(x, w_gate, W1, b1, gamma1, beta1, W2, b2):
    bsz, d = x.shape
    num_e = w_gate.shape[1]
    hdim = W1.shape[2]
    grid = bsz // BT
    xt = x.T

    logits = pl.pallas_call(
        _gating_body,
        grid=(bsz // BTG,),
        in_specs=[
            pl.BlockSpec((d, BTG), lambda i: (0, i)),
            pl.BlockSpec((num_e, d), lambda i: (0, 0)),
        ],
        out_specs=pl.BlockSpec((BTG, num_e), lambda i: (i, 0)),
        out_shape=jax.ShapeDtypeStruct((bsz, num_e), jnp.float32),
        compiler_params=pltpu.CompilerParams(
            dimension_semantics=("arbitrary",)),
    )(xt, w_gate.T)

    gates = _make_gate_sc(bsz, num_e)(
        logits.T.reshape(num_e * bsz)).reshape(num_e, bsz).T

    w2s = W2[:, :, 0]
    b2r = b2.reshape(1, num_e)

    o = pl.pallas_call(
        _moe_body,
        grid=(grid,),
        in_specs=[
            pl.BlockSpec((d, BT), lambda i: (0, i)),
            pl.BlockSpec((num_e, d, hdim), lambda i: (0, 0, 0)),
            pl.BlockSpec((num_e, hdim), lambda i: (0, 0)),
            pl.BlockSpec((num_e, hdim), lambda i: (0, 0)),
            pl.BlockSpec((num_e, hdim), lambda i: (0, 0)),
            pl.BlockSpec((num_e, hdim), lambda i: (0, 0)),
            pl.BlockSpec((1, num_e), lambda i: (0, 0)),
        ],
        out_specs=pl.BlockSpec((BT, num_e), lambda i: (i, 0)),
        out_shape=jax.ShapeDtypeStruct((bsz, num_e), jnp.float32),
        scratch_shapes=[pltpu.VMEM((num_e, d, hdim), jnp.bfloat16)],
        compiler_params=pltpu.CompilerParams(
            dimension_semantics=("arbitrary",)),
    )(xt, W1, b1, gamma1, beta1, w2s, b2r)

    y = pl.pallas_call(
        _combine_body,
        in_specs=[
            pl.BlockSpec((bsz, num_e), lambda: (0, 0)),
            pl.BlockSpec((bsz, num_e), lambda: (0, 0)),
        ],
        out_specs=pl.BlockSpec((bsz, 1), lambda: (0, 0)),
        out_shape=jax.ShapeDtypeStruct((bsz, 1), jnp.float32),
    )(gates, o)

    return y


# speedup vs baseline: 1.5610x; 1.0543x over previous
import functools

import jax
import jax.numpy as jnp
from jax import lax
from jax.experimental import pallas as pl
from jax.experimental.pallas import tpu as pltpu
from jax.experimental.pallas import tpu_sc as plsc

K = 2
BT = 1024
NEG = -3e38
DIMS_T = (((0,), (0,)), ((), ()))
DIMS_TT = (((0,), (1,)), ((), ()))


def _moe_body(xt_ref, wgt_ref, w1_ref, b1_ref, g1_ref, be1_ref, w2_ref,
              b2_ref, logits_ref, o_ref, w1bf_ref):
    i = pl.program_id(0)

    @pl.when(i == 0)
    def _():
        w1bf_ref[...] = w1_ref[...].astype(jnp.bfloat16)

    xt = xt_ref[...]
    xh = xt.astype(jnp.bfloat16)
    xl = (xt - xh.astype(jnp.float32)).astype(jnp.bfloat16)

    wg = wgt_ref[...]
    wh = wg.astype(jnp.bfloat16)
    wl = (wg - wh.astype(jnp.float32)).astype(jnp.bfloat16)
    acc = lax.dot_general(xh, wh, DIMS_TT, preferred_element_type=jnp.float32)
    acc = acc + lax.dot_general(xl, wh, DIMS_TT,
                                preferred_element_type=jnp.float32)
    acc = acc + lax.dot_general(xh, wl, DIMS_TT,
                                preferred_element_type=jnp.float32)
    logits_ref[...] = acc

    num_e = w1_ref.shape[0]
    cols = []
    for e in range(num_e):
        h = lax.dot_general(
            xh, w1bf_ref[e], DIMS_T,
            preferred_element_type=jnp.float32)
        h = h + b1_ref[e:e + 1, :]
        mu = jnp.mean(h, axis=1, keepdims=True)
        d = h - mu
        var = jnp.mean(d * d, axis=1, keepdims=True)
        hn = d * lax.rsqrt(var + 1e-5) * g1_ref[e:e + 1, :] + be1_ref[e:e + 1, :]
        hr = jnp.maximum(hn, 0.0)
        z = jnp.sum(hr * w2_ref[e:e + 1, :], axis=1, keepdims=True)
        cols.append(jax.nn.sigmoid(z + b2_ref[0:1, e:e + 1]))
    o_ref[...] = jnp.concatenate(cols, axis=1)


def _combine_body(gt_ref, o_ref, y_ref):
    ot = jnp.transpose(o_ref[...])
    y_ref[...] = jnp.sum(gt_ref[...] * ot, axis=0, keepdims=True)


def _make_gate_sc(bsz, num_e):
    info = plsc.get_sparse_core_info()
    nc, ns, nl = info.num_cores, info.num_subcores, info.num_lanes
    nw = nc * ns
    chunk = bsz // nw
    mesh = plsc.VectorSubcoreMesh(core_axis_name="c", subcore_axis_name="s")

    @functools.partial(
        pl.kernel, mesh=mesh,
        out_type=jax.ShapeDtypeStruct((num_e * bsz,), jnp.float32),
        scratch_types=[pltpu.VMEM((num_e * chunk,), jnp.float32),
                       pltpu.VMEM((num_e * chunk,), jnp.float32)],
    )
    def gate_sc(logits_hbm, gates_hbm, lg_v, out_v):
        wid = lax.axis_index("s") * nc + lax.axis_index("c")
        base = wid * chunk
        for e in range(num_e):
            pltpu.sync_copy(logits_hbm.at[pl.ds(e * bsz + base, chunk)],
                            lg_v.at[pl.ds(e * chunk, chunk)])
        for g in range(chunk // nl):
            l = [lg_v[pl.ds(e * chunk + g * nl, nl)] for e in range(num_e)]
            m1 = l[0]
            for e in range(1, num_e):
                m1 = jnp.maximum(m1, l[e])
            idx1 = jnp.full((nl,), num_e - 1, jnp.int32)
            for e in range(num_e - 2, -1, -1):
                idx1 = jnp.where(l[e] == m1, e, idx1)
            lm = [jnp.where(idx1 == e, NEG, l[e]) for e in range(num_e)]
            m2 = lm[0]
            for e in range(1, num_e):
                m2 = jnp.maximum(m2, lm[e])
            idx2 = jnp.full((nl,), num_e - 1, jnp.int32)
            for e in range(num_e - 2, -1, -1):
                idx2 = jnp.where(lm[e] == m2, e, idx2)
            s = jnp.exp(l[0] - m1)
            for e in range(1, num_e):
                s = s + jnp.exp(l[e] - m1)
            p1 = 1.0 / s
            p2 = jnp.exp(m2 - m1) / s
            denom = p1 + p2 + 1e-6
            g1 = p1 / denom
            g2 = p2 / denom
            zero = jnp.zeros((nl,), jnp.float32)
            for e in range(num_e):
                ge = jnp.where(idx1 == e, g1, jnp.where(idx2 == e, g2, zero))
                out_v[pl.ds(e * chunk + g * nl, nl)] = ge
        for e in range(num_e):
            pltpu.sync_copy(out_v.at[pl.ds(e * chunk, chunk)],
                            gates_hbm.at[pl.ds(e * bsz + base, chunk)])

    return gate_sc


def kernel(x, w_gate, W1, b1, gamma1, beta1, W2, b2):
    bsz, d = x.shape
    num_e = w_gate.shape[1]
    hdim = W1.shape[2]
    grid = bsz // BT
    xt = x.T

    logits, o = pl.pallas_call(
        _moe_body,
        grid=(grid,),
        in_specs=[
            pl.BlockSpec((d, BT), lambda i: (0, i)),
            pl.BlockSpec((num_e, d), lambda i: (0, 0)),
            pl.BlockSpec((num_e, d, hdim), lambda i: (0, 0, 0)),
            pl.BlockSpec((num_e, hdim), lambda i: (0, 0)),
            pl.BlockSpec((num_e, hdim), lambda i: (0, 0)),
            pl.BlockSpec((num_e, hdim), lambda i: (0, 0)),
            pl.BlockSpec((num_e, hdim), lambda i: (0, 0)),
            pl.BlockSpec((1, num_e), lambda i: (0, 0)),
        ],
        out_specs=[
            pl.BlockSpec((BT, num_e), lambda i: (i, 0)),
            pl.BlockSpec((BT, num_e), lambda i: (i, 0)),
        ],
        out_shape=[
            jax.ShapeDtypeStruct((bsz, num_e), jnp.float32),
            jax.ShapeDtypeStruct((bsz, num_e), jnp.float32),
        ],
        scratch_shapes=[pltpu.VMEM((num_e, d, hdim), jnp.bfloat16)],
        compiler_params=pltpu.CompilerParams(
            dimension_semantics=("arbitrary",)),
    )(xt, w_gate.T, W1, b1, gamma1, beta1, W2[:, :, 0], b2.reshape(1, num_e))

    gates_t = _make_gate_sc(bsz, num_e)(
        logits.T.reshape(num_e * bsz)).reshape(num_e, bsz)

    y_row = pl.pallas_call(
        _combine_body,
        in_specs=[
            pl.BlockSpec((num_e, bsz), lambda: (0, 0)),
            pl.BlockSpec((bsz, num_e), lambda: (0, 0)),
        ],
        out_specs=pl.BlockSpec((1, bsz), lambda: (0, 0)),
        out_shape=jax.ShapeDtypeStruct((1, bsz), jnp.float32),
    )(gates_t, o)

    return y_row.reshape(bsz, 1)
